# baseline (speedup 1.0000x reference)
import functools

import jax
import jax.numpy as jnp
from jax import lax
from jax.experimental import pallas as pl
from jax.experimental.pallas import tpu as pltpu
from jax.experimental.pallas import tpu_sc as plsc

_B, _T, _F, _J = 16384, 200, 4, 15
_NW = 32
_BW = _B // _NW
_BC = 256
_TT = 8
_NTT = _T // _TT
_NCH = _NTT * (_BW // _BC)
_L = 16
_VPC = _TT * _BC // _L

_mesh = plsc.VectorSubcoreMesh(core_axis_name="c", subcore_axis_name="s")


@functools.partial(
    pl.kernel,
    mesh=_mesh,
    out_type=jax.ShapeDtypeStruct((_J, _T, _B), jnp.float32),
    compiler_params=pltpu.CompilerParams(
        needs_layout_passes=False, use_tc_tiling_on_sc=True),
    scratch_types=[
        pltpu.VMEM((_TT, _F, _BC), jnp.int32),
        pltpu.VMEM((_TT, _F, _BC), jnp.int32),
        pltpu.VMEM((_J, _TT, _BC), jnp.float32),
        pltpu.VMEM((_J, _TT, _BC), jnp.float32),
        pltpu.VMEM((11, 3), jnp.float32),
        pltpu.VMEM((18, 4), jnp.float32),
        pltpu.VMEM((24, 3), jnp.float32),
        pltpu.VMEM((7, 5), jnp.float32),
        pltpu.SemaphoreType.DMA,
        pltpu.SemaphoreType.DMA,
        pltpu.SemaphoreType.DMA,
        pltpu.SemaphoreType.DMA,
    ],
)
def _embed(x_hbm, w1_hbm, w2_hbm, w3_hbm, w4_hbm, out_hbm,
           xv0, xv1, ov0, ov1, w1v, w2v, w3v, w4v,
           si0, si1, so0, so1):
    cid = lax.axis_index("c")
    sid = lax.axis_index("s")
    wid = sid * 2 + cid
    pltpu.sync_copy(w1_hbm, w1v)
    pltpu.sync_copy(w2_hbm, w2v)
    pltpu.sync_copy(w3_hbm, w3v)
    pltpu.sync_copy(w4_hbm, w4v)
    b_w = wid * _BW
    nbs = _BW // _BC

    xvs = (xv0, xv1)
    ovs = (ov0, ov1)
    sis = (si0, si1)
    sos = (so0, so1)
    zeros = jnp.zeros((_L,), jnp.int32)

    def chunk_slices(gi):
        t0 = (gi // nbs) * _TT
        b0 = b_w + (gi % nbs) * _BC
        return t0, b0

    def start_in(gi, b):
        t0, b0 = chunk_slices(gi)
        pltpu.async_copy(
            x_hbm.at[pl.ds(t0, _TT), :, pl.ds(b0, _BC)], xvs[b], sis[b])

    def wait_in(b):
        pltpu.make_async_copy(
            x_hbm.at[pl.ds(0, _TT), :, pl.ds(0, _BC)], xvs[b], sis[b]).wait()

    def start_out(gi, b):
        t0, b0 = chunk_slices(gi)
        pltpu.async_copy(
            ovs[b], out_hbm.at[:, pl.ds(t0, _TT), pl.ds(b0, _BC)], sos[b])

    def wait_out(b):
        pltpu.make_async_copy(
            ovs[b], out_hbm.at[:, pl.ds(0, _TT), pl.ds(0, _BC)], sos[b]).wait()

    def compute(b):
        xv = xvs[b]
        ov = ovs[b]

        for t in range(_TT):

            @plsc.parallel_loop(0, _BC, _L, unroll=4)
            def bv_body(boff):
                a = xv[t, 0, pl.ds(boff, _L)]
                bb = xv[t, 1, pl.ds(boff, _L)]
                cc = xv[t, 2, pl.ds(boff, _L)]
                dd = xv[t, 3, pl.ds(boff, _L)]
                a = jnp.minimum(jnp.maximum(a, 0), 10)
                bb = jnp.minimum(jnp.maximum(bb, 0), 17)
                cc = jnp.minimum(jnp.maximum(cc, 0), 23)
                dd = jnp.minimum(jnp.maximum(dd, 0), 6)
                for j in range(3):
                    ov[j, t, pl.ds(boff, _L)] = plsc.load_gather(
                        w1v, [a, zeros + j])
                for j in range(4):
                    ov[3 + j, t, pl.ds(boff, _L)] = plsc.load_gather(
                        w2v, [bb, zeros + j])
                for j in range(3):
                    ov[7 + j, t, pl.ds(boff, _L)] = plsc.load_gather(
                        w3v, [cc, zeros + j])
                for j in range(5):
                    ov[10 + j, t, pl.ds(boff, _L)] = plsc.load_gather(
                        w4v, [dd, zeros + j])

    start_in(0, 0)

    def pair_body(gp, carry):
        for b in (0, 1):
            gi = gp * 2 + b

            @pl.when(gi + 1 < _NCH)
            def _():
                start_in(gi + 1, 1 - b)

            wait_in(b)

            @pl.when(gi >= 2)
            def _():
                wait_out(b)

            compute(b)
            start_out(gi, b)
        return carry

    lax.fori_loop(0, _NCH // 2, pair_body, 0)
    wait_out(0)
    wait_out(1)


def kernel(x, W_wdir, W_weather, W_hour, W_weekday):
    xt = jnp.transpose(x.astype(jnp.int32), (1, 2, 0))
    out_t = _embed(xt, W_wdir, W_weather, W_hour, W_weekday)
    return jnp.transpose(out_t, (2, 1, 0))


# speedup vs baseline: 1.0006x; 1.0006x over previous
import functools

import jax
import jax.numpy as jnp
from jax import lax
from jax.experimental import pallas as pl
from jax.experimental.pallas import tpu as pltpu
from jax.experimental.pallas import tpu_sc as plsc

_B, _T, _F, _J = 16384, 200, 4, 15
_NW = 32
_BW = _B // _NW
_BC = 256
_TT = 8
_NTT = _T // _TT
_NCH = _NTT * (_BW // _BC)
_L = 16
_VPC = _TT * _BC // _L

_mesh = plsc.VectorSubcoreMesh(core_axis_name="c", subcore_axis_name="s")


@functools.partial(
    pl.kernel,
    mesh=_mesh,
    out_type=jax.ShapeDtypeStruct((_J, _T, _B), jnp.float32),
    compiler_params=pltpu.CompilerParams(
        needs_layout_passes=False, use_tc_tiling_on_sc=True),
    scratch_types=[
        pltpu.VMEM((_TT, _F, _BC), jnp.int32),
        pltpu.VMEM((_TT, _F, _BC), jnp.int32),
        pltpu.VMEM((_J, _TT, _BC), jnp.float32),
        pltpu.VMEM((_J, _TT, _BC), jnp.float32),
        pltpu.VMEM((11, 3), jnp.float32),
        pltpu.VMEM((18, 4), jnp.float32),
        pltpu.VMEM((24, 3), jnp.float32),
        pltpu.VMEM((7, 5), jnp.float32),
        pltpu.SemaphoreType.DMA,
        pltpu.SemaphoreType.DMA,
        pltpu.SemaphoreType.DMA,
        pltpu.SemaphoreType.DMA,
    ],
)
def _embed(x_hbm, w1_hbm, w2_hbm, w3_hbm, w4_hbm, out_hbm,
           xv0, xv1, ov0, ov1, w1v, w2v, w3v, w4v,
           si0, si1, so0, so1):
    cid = lax.axis_index("c")
    sid = lax.axis_index("s")
    wid = sid * 2 + cid
    pltpu.sync_copy(w1_hbm, w1v)
    pltpu.sync_copy(w2_hbm, w2v)
    pltpu.sync_copy(w3_hbm, w3v)
    pltpu.sync_copy(w4_hbm, w4v)
    b_w = wid * _BW
    nbs = _BW // _BC

    xvs = (xv0, xv1)
    ovs = (ov0, ov1)
    sis = (si0, si1)
    sos = (so0, so1)
    zeros = jnp.zeros((_L,), jnp.int32)

    def chunk_slices(gi):
        t0 = (gi // nbs) * _TT
        b0 = b_w + (gi % nbs) * _BC
        return t0, b0

    def start_in(gi, b):
        t0, b0 = chunk_slices(gi)
        pltpu.async_copy(
            x_hbm.at[pl.ds(t0, _TT), :, pl.ds(b0, _BC)], xvs[b], sis[b])

    def wait_in(b):
        pltpu.make_async_copy(
            x_hbm.at[pl.ds(0, _TT), :, pl.ds(0, _BC)], xvs[b], sis[b]).wait()

    def start_out(gi, b):
        t0, b0 = chunk_slices(gi)
        pltpu.async_copy(
            ovs[b], out_hbm.at[:, pl.ds(t0, _TT), pl.ds(b0, _BC)], sos[b])

    def wait_out(b):
        pltpu.make_async_copy(
            ovs[b], out_hbm.at[:, pl.ds(0, _TT), pl.ds(0, _BC)], sos[b]).wait()

    def compute(b):
        xv = xvs[b]
        ov = ovs[b]

        def t_body(t, carry):

            @plsc.parallel_loop(0, _BC, _L, unroll=4)
            def bv_body(boff):
                a = xv[t, 0, pl.ds(boff, _L)]
                bb = xv[t, 1, pl.ds(boff, _L)]
                cc = xv[t, 2, pl.ds(boff, _L)]
                dd = xv[t, 3, pl.ds(boff, _L)]
                a = jnp.minimum(jnp.maximum(a, 0), 10)
                bb = jnp.minimum(jnp.maximum(bb, 0), 17)
                cc = jnp.minimum(jnp.maximum(cc, 0), 23)
                dd = jnp.minimum(jnp.maximum(dd, 0), 6)
                for j in range(3):
                    ov[j, t, pl.ds(boff, _L)] = plsc.load_gather(
                        w1v, [a, zeros + j])
                for j in range(4):
                    ov[3 + j, t, pl.ds(boff, _L)] = plsc.load_gather(
                        w2v, [bb, zeros + j])
                for j in range(3):
                    ov[7 + j, t, pl.ds(boff, _L)] = plsc.load_gather(
                        w3v, [cc, zeros + j])
                for j in range(5):
                    ov[10 + j, t, pl.ds(boff, _L)] = plsc.load_gather(
                        w4v, [dd, zeros + j])

            return carry

        lax.fori_loop(0, _TT, t_body, 0)

    start_in(0, 0)

    def pair_body(gp, carry):
        for b in (0, 1):
            gi = gp * 2 + b

            @pl.when(gi + 1 < _NCH)
            def _():
                start_in(gi + 1, 1 - b)

            wait_in(b)

            @pl.when(gi >= 2)
            def _():
                wait_out(b)

            compute(b)
            start_out(gi, b)
        return carry

    lax.fori_loop(0, _NCH // 2, pair_body, 0)
    wait_out(0)
    wait_out(1)


def kernel(x, W_wdir, W_weather, W_hour, W_weekday):
    xt = jnp.transpose(x.astype(jnp.int32), (1, 2, 0))
    out_t = _embed(xt, W_wdir, W_weather, W_hour, W_weekday)
    return jnp.transpose(out_t, (2, 1, 0))


# speedup vs baseline: 1.0666x; 1.0660x over previous
import functools

import jax
import jax.numpy as jnp
from jax import lax
from jax.experimental import pallas as pl
from jax.experimental.pallas import tpu as pltpu
from jax.experimental.pallas import tpu_sc as plsc

_B, _T, _F, _J = 16384, 200, 4, 15
_NW = 32
_BW = _B // _NW
_BC = 256
_TT = 8
_NTT = _T // _TT
_NCH = _NTT * (_BW // _BC)
_L = 16
_VPC = _TT * _BC // _L

_mesh = plsc.VectorSubcoreMesh(core_axis_name="c", subcore_axis_name="s")


@functools.partial(
    pl.kernel,
    mesh=_mesh,
    out_type=jax.ShapeDtypeStruct((_J, _T, _B), jnp.float32),
    compiler_params=pltpu.CompilerParams(
        needs_layout_passes=False, use_tc_tiling_on_sc=True),
    scratch_types=[
        pltpu.VMEM((_TT, _F, _BC), jnp.int32),
        pltpu.VMEM((_TT, _F, _BC), jnp.int32),
        pltpu.VMEM((_J, _TT, _BC), jnp.float32),
        pltpu.VMEM((_J, _TT, _BC), jnp.float32),
        pltpu.VMEM((11, 3), jnp.float32),
        pltpu.VMEM((18, 4), jnp.float32),
        pltpu.VMEM((24, 3), jnp.float32),
        pltpu.VMEM((7, 5), jnp.float32),
        pltpu.SemaphoreType.DMA,
        pltpu.SemaphoreType.DMA,
        pltpu.SemaphoreType.DMA,
        pltpu.SemaphoreType.DMA,
    ],
)
def _embed(x_hbm, w1_hbm, w2_hbm, w3_hbm, w4_hbm, out_hbm,
           xv0, xv1, ov0, ov1, w1v, w2v, w3v, w4v,
           si0, si1, so0, so1):
    cid = lax.axis_index("c")
    sid = lax.axis_index("s")
    wid = sid * 2 + cid
    pltpu.sync_copy(w1_hbm, w1v)
    pltpu.sync_copy(w2_hbm, w2v)
    pltpu.sync_copy(w3_hbm, w3v)
    pltpu.sync_copy(w4_hbm, w4v)
    b_w = wid * _BW
    nbs = _BW // _BC

    xvs = (xv0, xv1)
    ovs = (ov0, ov1)
    sis = (si0, si1)
    sos = (so0, so1)
    zeros = jnp.zeros((_L,), jnp.int32)

    def chunk_slices(gi):
        t0 = (gi // nbs) * _TT
        b0 = b_w + (gi % nbs) * _BC
        return t0, b0

    def start_in(gi, b):
        t0, b0 = chunk_slices(gi)
        pltpu.async_copy(
            x_hbm.at[pl.ds(t0, _TT), :, pl.ds(b0, _BC)], xvs[b], sis[b])

    def wait_in(b):
        pltpu.make_async_copy(
            x_hbm.at[pl.ds(0, _TT), :, pl.ds(0, _BC)], xvs[b], sis[b]).wait()

    def start_out(gi, b):
        t0, b0 = chunk_slices(gi)
        pltpu.async_copy(
            ovs[b], out_hbm.at[:, pl.ds(t0, _TT), pl.ds(b0, _BC)], sos[b])

    def wait_out(b):
        pltpu.make_async_copy(
            ovs[b], out_hbm.at[:, pl.ds(0, _TT), pl.ds(0, _BC)], sos[b]).wait()

    def compute(b):
        xv = xvs[b]
        ov = ovs[b]

        def t_body(t, carry):

            @plsc.parallel_loop(0, _BC, _L, unroll=2)
            def bv_body(boff):
                a = xv[t, 0, pl.ds(boff, _L)]
                bb = xv[t, 1, pl.ds(boff, _L)]
                cc = xv[t, 2, pl.ds(boff, _L)]
                dd = xv[t, 3, pl.ds(boff, _L)]
                a = jnp.minimum(jnp.maximum(a, 0), 10)
                bb = jnp.minimum(jnp.maximum(bb, 0), 17)
                cc = jnp.minimum(jnp.maximum(cc, 0), 23)
                dd = jnp.minimum(jnp.maximum(dd, 0), 6)
                vals = []
                for j in range(3):
                    vals.append(plsc.load_gather(w1v, [a, zeros + j]))
                for j in range(4):
                    vals.append(plsc.load_gather(w2v, [bb, zeros + j]))
                for j in range(3):
                    vals.append(plsc.load_gather(w3v, [cc, zeros + j]))
                for j in range(5):
                    vals.append(plsc.load_gather(w4v, [dd, zeros + j]))
                for j in range(15):
                    ov[j, t, pl.ds(boff, _L)] = vals[j]

            return carry

        lax.fori_loop(0, _TT, t_body, 0)

    start_in(0, 0)

    def pair_body(gp, carry):
        for b in (0, 1):
            gi = gp * 2 + b

            @pl.when(gi + 1 < _NCH)
            def _():
                start_in(gi + 1, 1 - b)

            wait_in(b)

            @pl.when(gi >= 2)
            def _():
                wait_out(b)

            compute(b)
            start_out(gi, b)
        return carry

    lax.fori_loop(0, _NCH // 2, pair_body, 0)
    wait_out(0)
    wait_out(1)


def kernel(x, W_wdir, W_weather, W_hour, W_weekday):
    xt = jnp.transpose(x.astype(jnp.int32), (1, 2, 0))
    out_t = _embed(xt, W_wdir, W_weather, W_hour, W_weekday)
    return jnp.transpose(out_t, (2, 1, 0))


# speedup vs baseline: 5.0745x; 4.7577x over previous
import functools

import jax
import jax.numpy as jnp
from jax import lax
from jax.experimental import pallas as pl
from jax.experimental.pallas import tpu as pltpu
from jax.experimental.pallas import tpu_sc as plsc

_B, _T, _F, _J = 16384, 200, 4, 15
_NW = 32
_BW = _B // _NW
_BC = 256
_TT = 8
_NTT = _T // _TT
_NCH = _NTT * (_BW // _BC)
_L = 16
_VPC = _TT * _BC // _L

_mesh = plsc.VectorSubcoreMesh(core_axis_name="c", subcore_axis_name="s")


@functools.partial(
    pl.kernel,
    mesh=_mesh,
    out_type=jax.ShapeDtypeStruct((_J, _T, _B), jnp.float32),
    compiler_params=pltpu.CompilerParams(
        needs_layout_passes=False, use_tc_tiling_on_sc=True),
    scratch_types=[
        pltpu.VMEM((_TT, _F, _BC), jnp.int32),
        pltpu.VMEM((_TT, _F, _BC), jnp.int32),
        pltpu.VMEM((_J, _TT, _BC), jnp.float32),
        pltpu.VMEM((_J, _TT, _BC), jnp.float32),
        pltpu.VMEM((33,), jnp.float32),
        pltpu.VMEM((72,), jnp.float32),
        pltpu.VMEM((72,), jnp.float32),
        pltpu.VMEM((35,), jnp.float32),
        pltpu.SemaphoreType.DMA,
        pltpu.SemaphoreType.DMA,
        pltpu.SemaphoreType.DMA,
        pltpu.SemaphoreType.DMA,
    ],
)
def _embed(x_hbm, w1_hbm, w2_hbm, w3_hbm, w4_hbm, out_hbm,
           xv0, xv1, ov0, ov1, w1v, w2v, w3v, w4v,
           si0, si1, so0, so1):
    cid = lax.axis_index("c")
    sid = lax.axis_index("s")
    wid = sid * 2 + cid
    pltpu.sync_copy(w1_hbm, w1v)
    pltpu.sync_copy(w2_hbm, w2v)
    pltpu.sync_copy(w3_hbm, w3v)
    pltpu.sync_copy(w4_hbm, w4v)
    b_w = wid * _BW
    nbs = _BW // _BC

    xvs = (xv0, xv1)
    ovs = (ov0, ov1)
    sis = (si0, si1)
    sos = (so0, so1)
    zeros = jnp.zeros((_L,), jnp.int32)

    def chunk_slices(gi):
        t0 = (gi // nbs) * _TT
        b0 = b_w + (gi % nbs) * _BC
        return t0, b0

    def start_in(gi, b):
        t0, b0 = chunk_slices(gi)
        pltpu.async_copy(
            x_hbm.at[pl.ds(t0, _TT), :, pl.ds(b0, _BC)], xvs[b], sis[b])

    def wait_in(b):
        pltpu.make_async_copy(
            x_hbm.at[pl.ds(0, _TT), :, pl.ds(0, _BC)], xvs[b], sis[b]).wait()

    def start_out(gi, b):
        t0, b0 = chunk_slices(gi)
        pltpu.async_copy(
            ovs[b], out_hbm.at[:, pl.ds(t0, _TT), pl.ds(b0, _BC)], sos[b])

    def wait_out(b):
        pltpu.make_async_copy(
            ovs[b], out_hbm.at[:, pl.ds(0, _TT), pl.ds(0, _BC)], sos[b]).wait()

    def compute(b):
        xv = xvs[b]
        ov = ovs[b]

        def t_body(t, carry):

            @plsc.parallel_loop(0, _BC, _L, unroll=2)
            def bv_body(boff):
                a = xv[t, 0, pl.ds(boff, _L)]
                bb = xv[t, 1, pl.ds(boff, _L)]
                cc = xv[t, 2, pl.ds(boff, _L)]
                dd = xv[t, 3, pl.ds(boff, _L)]
                a = jnp.minimum(jnp.maximum(a, 0), 10) * 3
                bb = jnp.minimum(jnp.maximum(bb, 0), 17) * 4
                cc = jnp.minimum(jnp.maximum(cc, 0), 23) * 3
                dd = jnp.minimum(jnp.maximum(dd, 0), 6) * 5
                vals = []
                for j in range(3):
                    vals.append(plsc.load_gather(w1v, [a + j]))
                for j in range(4):
                    vals.append(plsc.load_gather(w2v, [bb + j]))
                for j in range(3):
                    vals.append(plsc.load_gather(w3v, [cc + j]))
                for j in range(5):
                    vals.append(plsc.load_gather(w4v, [dd + j]))
                for j in range(15):
                    ov[j, t, pl.ds(boff, _L)] = vals[j]

            return carry

        lax.fori_loop(0, _TT, t_body, 0)

    start_in(0, 0)

    def pair_body(gp, carry):
        for b in (0, 1):
            gi = gp * 2 + b

            @pl.when(gi + 1 < _NCH)
            def _():
                start_in(gi + 1, 1 - b)

            wait_in(b)

            @pl.when(gi >= 2)
            def _():
                wait_out(b)

            compute(b)
            start_out(gi, b)
        return carry

    lax.fori_loop(0, _NCH // 2, pair_body, 0)
    wait_out(0)
    wait_out(1)


def kernel(x, W_wdir, W_weather, W_hour, W_weekday):
    xt = jnp.transpose(x.astype(jnp.int32), (1, 2, 0))
    out_t = _embed(xt, W_wdir.reshape(33), W_weather.reshape(72),
                   W_hour.reshape(72), W_weekday.reshape(35))
    return jnp.transpose(out_t, (2, 1, 0))
